# baseline (speedup 1.0000x reference)
import jax
import jax.numpy as jnp
import numpy as np
from jax import lax
from jax.experimental import pallas as pl
from jax.experimental.pallas import tpu as pltpu
from jax.experimental.pallas import tpu_sc as plsc

B = 16384
D = 13
F = 26
V = 100000
K = 16

NC = 2
NS = 16
NW = NC * NS
R = F * K
RPW = R // NW

BLK = 512


BH = B // 2


def _plane_gather(plane_v, idx_v, val_v):
    def body(i, carry):
        sl = pl.ds(i * 16, 16)
        val_v[sl] = plsc.load_gather(plane_v, [idx_v[sl]])
        return carry

    lax.fori_loop(0, BH // 16, body, 0, unroll=4)


def _sc_gather_body(s_hbm, tab_hbm, lint_hbm, emb_out, lin_out,
                    idx_v, val_v, plane_v):
    wid = lax.axis_index("s") * NC + lax.axis_index("c")
    r0 = wid * RPW
    for j in range(RPW):
        r = r0 + j
        pltpu.sync_copy(tab_hbm.at[r], plane_v)
        for h in range(2):
            pltpu.sync_copy(s_hbm.at[r // K, pl.ds(h * BH, BH)], idx_v)
            _plane_gather(plane_v, idx_v, val_v)
            pltpu.sync_copy(val_v, emb_out.at[r, pl.ds(h * BH, BH)])
    @pl.when(wid < F)
    def _():
        pltpu.sync_copy(lint_hbm.at[wid], plane_v)
        for h in range(2):
            pltpu.sync_copy(s_hbm.at[wid, pl.ds(h * BH, BH)], idx_v)
            _plane_gather(plane_v, idx_v, val_v)
            pltpu.sync_copy(val_v, lin_out.at[wid, pl.ds(h * BH, BH)])


def _sc_gather(sT, tab, linT):
    return pl.kernel(
        _sc_gather_body,
        out_type=(
            jax.ShapeDtypeStruct((R, B), jnp.float32),
            jax.ShapeDtypeStruct((F, B), jnp.float32),
        ),
        mesh=plsc.VectorSubcoreMesh(core_axis_name="c", subcore_axis_name="s",
                                    num_cores=NC, num_subcores=NS),
        scratch_types=[
            pltpu.VMEM((BH,), jnp.int32),
            pltpu.VMEM((BH,), jnp.float32),
            pltpu.VMEM((V,), jnp.float32),
        ],
        compiler_params=pltpu.CompilerParams(use_tc_tiling_on_sc=True, needs_layout_passes=False),
    )(sT, tab, linT)


def _tc_body(d_ref, emb_ref, lin_ref, wd_ref, wl_ref, g_ref,
             w1d_ref, w1s_ref, w2_ref, w3_ref, wo_ref, c_ref, out_ref):
    x = d_ref[...]
    e = emb_ref[...]
    f32 = jnp.float32
    de = jnp.dot(wd_ref[...], x, preferred_element_type=f32)
    lin = jnp.dot(wl_ref[...], x, preferred_element_type=f32)
    first = lin + jnp.sum(lin_ref[...], axis=0, keepdims=True)
    s = de + jnp.dot(g_ref[...], e, preferred_element_type=f32)
    sumsq = (jnp.sum(de * de, axis=0, keepdims=True)
             + jnp.sum(e * e, axis=0, keepdims=True))
    second = 0.5 * (jnp.sum(s * s, axis=0, keepdims=True) - sumsq)
    h = jnp.maximum(jnp.dot(w1d_ref[...], de, preferred_element_type=f32)
                    + jnp.dot(w1s_ref[...], e, preferred_element_type=f32),
                    0.0)
    h = jnp.maximum(jnp.dot(w2_ref[...], h, preferred_element_type=f32), 0.0)
    h = jnp.maximum(jnp.dot(w3_ref[...], h, preferred_element_type=f32), 0.0)
    deep = jnp.dot(wo_ref[...], h, preferred_element_type=f32)
    out_ref[...] = c_ref[...] + first + second + deep


def _tc_compute(dT, embT, linT, wd, wl, gT, w1d, w1s, w2, w3, wo, cconst):
    nblk = B // BLK

    def full(shape):
        return pl.BlockSpec(shape, lambda i: (0, 0))

    return pl.pallas_call(
        _tc_body,
        grid=(nblk,),
        in_specs=[
            pl.BlockSpec((D, BLK), lambda i: (0, i)),
            pl.BlockSpec((R, BLK), lambda i: (0, i)),
            pl.BlockSpec((F, BLK), lambda i: (0, i)),
            full((K, D)),
            full((1, D)),
            full((K, R)),
            full((512, K)),
            full((512, R)),
            full((256, 512)),
            full((128, 256)),
            full((1, 128)),
            full((1, 1)),
        ],
        out_specs=pl.BlockSpec((1, BLK), lambda i: (0, i)),
        out_shape=jax.ShapeDtypeStruct((1, B), jnp.float32),
    )(dT, embT, linT, wd, wl, gT, w1d, w1s, w2, w3, wo, cconst)


_GT_NP = np.tile(np.eye(K, dtype=np.float32), (1, F))


def kernel(d0, d1, d2, d3, d4, d5, d6, d7, d8, d9, d10, d11, d12,
           s0, s1, s2, s3, s4, s5, s6, s7, s8, s9, s10, s11, s12,
           s13, s14, s15, s16, s17, s18, s19, s20, s21, s22, s23, s24, s25,
           W_dense, W_lin, b_lin, emb_tables, lin_tables, W1, W2, W3, Wout,
           bias):
    dT = jnp.stack([d0, d1, d2, d3, d4, d5, d6, d7, d8, d9, d10, d11, d12],
                   axis=0)
    sT = jnp.stack([s0, s1, s2, s3, s4, s5, s6, s7, s8, s9, s10, s11, s12,
                    s13, s14, s15, s16, s17, s18, s19, s20, s21, s22, s23,
                    s24, s25], axis=0)
    tab = emb_tables.transpose(0, 2, 1).reshape(R, V)
    linT = lin_tables.transpose(0, 2, 1).reshape(F, V)

    embT, linsT = _sc_gather(sT, tab, linT)

    cconst = b_lin.reshape(1, 1) + bias.reshape(1, 1)
    outT = _tc_compute(dT, embT, linsT,
                       W_dense, W_lin, jnp.asarray(_GT_NP),
                       W1[:, :K], W1[:, K:], W2, W3, Wout,
                       cconst)
    return outT.reshape(B, 1)


# speedup vs baseline: 1.0910x; 1.0910x over previous
import jax
import jax.numpy as jnp
import numpy as np
from jax import lax
from jax.experimental import pallas as pl
from jax.experimental.pallas import tpu as pltpu
from jax.experimental.pallas import tpu_sc as plsc

B = 16384
D = 13
F = 26
V = 100000
K = 16

NC = 2
NS = 16
NW = NC * NS
R = F * K
RPW = R // NW

BLK = 512


BH = B // 2


def _plane_gather(plane_v, idx_v, off, val_v):
    def body(i, carry):
        val_v[pl.ds(i * 16, 16)] = plsc.load_gather(
            plane_v, [idx_v[pl.ds(off + i * 16, 16)]])
        return carry

    lax.fori_loop(0, BH // 16, body, 0, unroll=4)


def _sc_gather_body(s_hbm, tab_hbm, lint_hbm, emb_out, lin_out,
                    idx_v, val_v, plane_v):
    wid = lax.axis_index("s") * NC + lax.axis_index("c")
    r0 = wid * RPW
    for j in range(RPW):
        r = r0 + j
        f = r // K
        if j == 0:
            pltpu.sync_copy(s_hbm.at[f], idx_v)
        else:
            @pl.when(f != (r - 1) // K)
            def _():
                pltpu.sync_copy(s_hbm.at[f], idx_v)
        pltpu.sync_copy(tab_hbm.at[r], plane_v)
        for h in range(2):
            _plane_gather(plane_v, idx_v, h * BH, val_v)
            pltpu.sync_copy(val_v, emb_out.at[r, pl.ds(h * BH, BH)])
    @pl.when(wid < F)
    def _():
        pltpu.sync_copy(s_hbm.at[wid], idx_v)
        pltpu.sync_copy(lint_hbm.at[wid], plane_v)
        for h in range(2):
            _plane_gather(plane_v, idx_v, h * BH, val_v)
            pltpu.sync_copy(val_v, lin_out.at[wid, pl.ds(h * BH, BH)])


def _sc_gather(sT, tab, linT):
    return pl.kernel(
        _sc_gather_body,
        out_type=(
            jax.ShapeDtypeStruct((R, B), jnp.float32),
            jax.ShapeDtypeStruct((F, B), jnp.float32),
        ),
        mesh=plsc.VectorSubcoreMesh(core_axis_name="c", subcore_axis_name="s",
                                    num_cores=NC, num_subcores=NS),
        scratch_types=[
            pltpu.VMEM((B,), jnp.int32),
            pltpu.VMEM((BH,), jnp.float32),
            pltpu.VMEM((V,), jnp.float32),
        ],
        compiler_params=pltpu.CompilerParams(use_tc_tiling_on_sc=True, needs_layout_passes=False),
    )(sT, tab, linT)


def _tc_body(d_ref, emb_ref, lin_ref, wd_ref, wl_ref, g_ref,
             w1d_ref, w1s_ref, w2_ref, w3_ref, wo_ref, c_ref, out_ref):
    x = d_ref[...]
    e = emb_ref[...]
    f32 = jnp.float32
    de = jnp.dot(wd_ref[...], x, preferred_element_type=f32)
    lin = jnp.dot(wl_ref[...], x, preferred_element_type=f32)
    first = lin + jnp.sum(lin_ref[...], axis=0, keepdims=True)
    s = de + jnp.dot(g_ref[...], e, preferred_element_type=f32)
    sumsq = (jnp.sum(de * de, axis=0, keepdims=True)
             + jnp.sum(e * e, axis=0, keepdims=True))
    second = 0.5 * (jnp.sum(s * s, axis=0, keepdims=True) - sumsq)
    h = jnp.maximum(jnp.dot(w1d_ref[...], de, preferred_element_type=f32)
                    + jnp.dot(w1s_ref[...], e, preferred_element_type=f32),
                    0.0)
    h = jnp.maximum(jnp.dot(w2_ref[...], h, preferred_element_type=f32), 0.0)
    h = jnp.maximum(jnp.dot(w3_ref[...], h, preferred_element_type=f32), 0.0)
    deep = jnp.dot(wo_ref[...], h, preferred_element_type=f32)
    out_ref[...] = c_ref[...] + first + second + deep


def _tc_compute(dT, embT, linT, wd, wl, gT, w1d, w1s, w2, w3, wo, cconst):
    nblk = B // BLK

    def full(shape):
        return pl.BlockSpec(shape, lambda i: (0, 0))

    return pl.pallas_call(
        _tc_body,
        grid=(nblk,),
        in_specs=[
            pl.BlockSpec((D, BLK), lambda i: (0, i)),
            pl.BlockSpec((R, BLK), lambda i: (0, i)),
            pl.BlockSpec((F, BLK), lambda i: (0, i)),
            full((K, D)),
            full((1, D)),
            full((K, R)),
            full((512, K)),
            full((512, R)),
            full((256, 512)),
            full((128, 256)),
            full((1, 128)),
            full((1, 1)),
        ],
        out_specs=pl.BlockSpec((1, BLK), lambda i: (0, i)),
        out_shape=jax.ShapeDtypeStruct((1, B), jnp.float32),
    )(dT, embT, linT, wd, wl, gT, w1d, w1s, w2, w3, wo, cconst)


_GT_NP = np.tile(np.eye(K, dtype=np.float32), (1, F))


def kernel(d0, d1, d2, d3, d4, d5, d6, d7, d8, d9, d10, d11, d12,
           s0, s1, s2, s3, s4, s5, s6, s7, s8, s9, s10, s11, s12,
           s13, s14, s15, s16, s17, s18, s19, s20, s21, s22, s23, s24, s25,
           W_dense, W_lin, b_lin, emb_tables, lin_tables, W1, W2, W3, Wout,
           bias):
    dT = jnp.stack([d0, d1, d2, d3, d4, d5, d6, d7, d8, d9, d10, d11, d12],
                   axis=0)
    sT = jnp.stack([s0, s1, s2, s3, s4, s5, s6, s7, s8, s9, s10, s11, s12,
                    s13, s14, s15, s16, s17, s18, s19, s20, s21, s22, s23,
                    s24, s25], axis=0)
    tab = emb_tables.transpose(0, 2, 1).reshape(R, V)
    linT = lin_tables.transpose(0, 2, 1).reshape(F, V)

    embT, linsT = _sc_gather(sT, tab, linT)

    cconst = b_lin.reshape(1, 1) + bias.reshape(1, 1)
    outT = _tc_compute(dT, embT, linsT,
                       W_dense, W_lin, jnp.asarray(_GT_NP),
                       W1[:, :K], W1[:, K:], W2, W3, Wout,
                       cconst)
    return outT.reshape(B, 1)


# speedup vs baseline: 1.1574x; 1.0608x over previous
import jax
import jax.numpy as jnp
import numpy as np
from jax import lax
from jax.experimental import pallas as pl
from jax.experimental.pallas import tpu as pltpu
from jax.experimental.pallas import tpu_sc as plsc

B = 16384
D = 13
F = 26
V = 100000
K = 16

NC = 2
NS = 16
NW = NC * NS
R = F * K
RPW = R // NW

BLK = 2048


BH = B // 2


def _plane_gather(plane_v, idx_v, off, val_v):
    def body(i, carry):
        val_v[pl.ds(i * 16, 16)] = plsc.load_gather(
            plane_v, [idx_v[pl.ds(off + i * 16, 16)]])
        return carry

    lax.fori_loop(0, BH // 16, body, 0, unroll=8)


def _sc_gather_body(s_hbm, tab_hbm, lint_hbm, emb_out, lin_out,
                    idx_v, val_v, plane_v):
    wid = lax.axis_index("s") * NC + lax.axis_index("c")
    r0 = wid * RPW
    for j in range(RPW):
        r = r0 + j
        f = r // K
        if j == 0:
            pltpu.sync_copy(s_hbm.at[f], idx_v)
        else:
            @pl.when(f != (r - 1) // K)
            def _():
                pltpu.sync_copy(s_hbm.at[f], idx_v)
        pltpu.sync_copy(tab_hbm.at[r], plane_v)
        for h in range(2):
            _plane_gather(plane_v, idx_v, h * BH, val_v)
            pltpu.sync_copy(val_v, emb_out.at[r, pl.ds(h * BH, BH)])
    @pl.when(wid < F)
    def _():
        pltpu.sync_copy(s_hbm.at[wid], idx_v)
        pltpu.sync_copy(lint_hbm.at[wid], plane_v)
        for h in range(2):
            _plane_gather(plane_v, idx_v, h * BH, val_v)
            pltpu.sync_copy(val_v, lin_out.at[wid, pl.ds(h * BH, BH)])


def _sc_gather(sT, tab, linT):
    return pl.kernel(
        _sc_gather_body,
        out_type=(
            jax.ShapeDtypeStruct((R, B), jnp.float32),
            jax.ShapeDtypeStruct((F, B), jnp.float32),
        ),
        mesh=plsc.VectorSubcoreMesh(core_axis_name="c", subcore_axis_name="s",
                                    num_cores=NC, num_subcores=NS),
        scratch_types=[
            pltpu.VMEM((B,), jnp.int32),
            pltpu.VMEM((BH,), jnp.float32),
            pltpu.VMEM((V,), jnp.float32),
        ],
        compiler_params=pltpu.CompilerParams(use_tc_tiling_on_sc=True, needs_layout_passes=False),
    )(sT, tab, linT)


def _tc_body(d_ref, emb_ref, lin_ref, wd_ref, wl_ref, g_ref,
             w1d_ref, w1s_ref, w2_ref, w3_ref, wo_ref, c_ref, out_ref):
    x = d_ref[...]
    e = emb_ref[...]
    f32 = jnp.float32
    de = jnp.dot(wd_ref[...], x, preferred_element_type=f32)
    lin = jnp.dot(wl_ref[...], x, preferred_element_type=f32)
    first = lin + jnp.sum(lin_ref[...], axis=0, keepdims=True)
    s = de + jnp.dot(g_ref[...], e, preferred_element_type=f32)
    sumsq = (jnp.sum(de * de, axis=0, keepdims=True)
             + jnp.sum(e * e, axis=0, keepdims=True))
    second = 0.5 * (jnp.sum(s * s, axis=0, keepdims=True) - sumsq)
    h = jnp.maximum(jnp.dot(w1d_ref[...], de, preferred_element_type=f32)
                    + jnp.dot(w1s_ref[...], e, preferred_element_type=f32),
                    0.0)
    h = jnp.maximum(jnp.dot(w2_ref[...], h, preferred_element_type=f32), 0.0)
    h = jnp.maximum(jnp.dot(w3_ref[...], h, preferred_element_type=f32), 0.0)
    deep = jnp.dot(wo_ref[...], h, preferred_element_type=f32)
    out_ref[...] = c_ref[...] + first + second + deep


def _tc_compute(dT, embT, linT, wd, wl, gT, w1d, w1s, w2, w3, wo, cconst):
    nblk = B // BLK

    def full(shape):
        return pl.BlockSpec(shape, lambda i: (0, 0))

    return pl.pallas_call(
        _tc_body,
        grid=(nblk,),
        in_specs=[
            pl.BlockSpec((D, BLK), lambda i: (0, i)),
            pl.BlockSpec((R, BLK), lambda i: (0, i)),
            pl.BlockSpec((F, BLK), lambda i: (0, i)),
            full((K, D)),
            full((1, D)),
            full((K, R)),
            full((512, K)),
            full((512, R)),
            full((256, 512)),
            full((128, 256)),
            full((1, 128)),
            full((1, 1)),
        ],
        out_specs=pl.BlockSpec((1, BLK), lambda i: (0, i)),
        out_shape=jax.ShapeDtypeStruct((1, B), jnp.float32),
    )(dT, embT, linT, wd, wl, gT, w1d, w1s, w2, w3, wo, cconst)


_GT_NP = np.tile(np.eye(K, dtype=np.float32), (1, F))


def kernel(d0, d1, d2, d3, d4, d5, d6, d7, d8, d9, d10, d11, d12,
           s0, s1, s2, s3, s4, s5, s6, s7, s8, s9, s10, s11, s12,
           s13, s14, s15, s16, s17, s18, s19, s20, s21, s22, s23, s24, s25,
           W_dense, W_lin, b_lin, emb_tables, lin_tables, W1, W2, W3, Wout,
           bias):
    dT = jnp.stack([d0, d1, d2, d3, d4, d5, d6, d7, d8, d9, d10, d11, d12],
                   axis=0)
    sT = jnp.stack([s0, s1, s2, s3, s4, s5, s6, s7, s8, s9, s10, s11, s12,
                    s13, s14, s15, s16, s17, s18, s19, s20, s21, s22, s23,
                    s24, s25], axis=0)
    tab = emb_tables.transpose(0, 2, 1).reshape(R, V)
    linT = lin_tables.transpose(0, 2, 1).reshape(F, V)

    embT, linsT = _sc_gather(sT, tab, linT)

    cconst = b_lin.reshape(1, 1) + bias.reshape(1, 1)
    outT = _tc_compute(dT, embT, linsT,
                       W_dense, W_lin, jnp.asarray(_GT_NP),
                       W1[:, :K], W1[:, K:], W2, W3, Wout,
                       cconst)
    return outT.reshape(B, 1)


# speedup vs baseline: 1.1761x; 1.0162x over previous
import jax
import jax.numpy as jnp
import numpy as np
from jax import lax
from jax.experimental import pallas as pl
from jax.experimental.pallas import tpu as pltpu
from jax.experimental.pallas import tpu_sc as plsc

B = 16384
D = 13
F = 26
V = 100000
K = 16

NC = 2
NS = 16
NW = NC * NS
R = F * K
RPW = R // NW

BLK = 2048


BQ = B // 4


def _plane_gather(plane_v, idx_v, off, val_v):
    def body(i, carry):
        val_v[pl.ds(i * 16, 16)] = plsc.load_gather(
            plane_v, [idx_v[pl.ds(off + i * 16, 16)]])
        return carry

    lax.fori_loop(0, BQ // 16, body, 0, unroll=8)


def _gather_plane_to(plane_v, idx_v, vals, sems, out_row):
    cps = [None, None]
    for q in range(4):
        if q >= 2:
            cps[q % 2].wait()
        _plane_gather(plane_v, idx_v, q * BQ, vals[q % 2])
        cps[q % 2] = pltpu.async_copy(
            vals[q % 2], out_row.at[pl.ds(q * BQ, BQ)], sems[q % 2])
    cps[0].wait()
    cps[1].wait()


def _sc_gather_body(s_hbm, tab_hbm, lint_hbm, emb_out, lin_out,
                    idx_v, val0_v, val1_v, plane_v, sem0, sem1):
    vals = (val0_v, val1_v)
    sems = (sem0, sem1)
    wid = lax.axis_index("s") * NC + lax.axis_index("c")
    r0 = wid * RPW
    for j in range(RPW):
        r = r0 + j
        f = r // K
        if j == 0:
            pltpu.sync_copy(s_hbm.at[f], idx_v)
        else:
            @pl.when(f != (r - 1) // K)
            def _():
                pltpu.sync_copy(s_hbm.at[f], idx_v)
        pltpu.sync_copy(tab_hbm.at[r], plane_v)
        _gather_plane_to(plane_v, idx_v, vals, sems, emb_out.at[r])
    @pl.when(wid < F)
    def _():
        pltpu.sync_copy(s_hbm.at[wid], idx_v)
        pltpu.sync_copy(lint_hbm.at[wid], plane_v)
        _gather_plane_to(plane_v, idx_v, vals, sems, lin_out.at[wid])


def _sc_gather(sT, tab, linT):
    return pl.kernel(
        _sc_gather_body,
        out_type=(
            jax.ShapeDtypeStruct((R, B), jnp.float32),
            jax.ShapeDtypeStruct((F, B), jnp.float32),
        ),
        mesh=plsc.VectorSubcoreMesh(core_axis_name="c", subcore_axis_name="s",
                                    num_cores=NC, num_subcores=NS),
        scratch_types=[
            pltpu.VMEM((B,), jnp.int32),
            pltpu.VMEM((BQ,), jnp.float32),
            pltpu.VMEM((BQ,), jnp.float32),
            pltpu.VMEM((V,), jnp.float32),
            pltpu.SemaphoreType.DMA,
            pltpu.SemaphoreType.DMA,
        ],
        compiler_params=pltpu.CompilerParams(use_tc_tiling_on_sc=True, needs_layout_passes=False),
    )(sT, tab, linT)


def _tc_body(d_ref, emb_ref, lin_ref, wd_ref, wl_ref, g_ref,
             w1d_ref, w1s_ref, w2_ref, w3_ref, wo_ref, c_ref, out_ref):
    x = d_ref[...]
    e = emb_ref[...]
    f32 = jnp.float32
    de = jnp.dot(wd_ref[...], x, preferred_element_type=f32)
    lin = jnp.dot(wl_ref[...], x, preferred_element_type=f32)
    first = lin + jnp.sum(lin_ref[...], axis=0, keepdims=True)
    s = de + jnp.dot(g_ref[...], e, preferred_element_type=f32)
    sumsq = (jnp.sum(de * de, axis=0, keepdims=True)
             + jnp.sum(e * e, axis=0, keepdims=True))
    second = 0.5 * (jnp.sum(s * s, axis=0, keepdims=True) - sumsq)
    h = jnp.maximum(jnp.dot(w1d_ref[...], de, preferred_element_type=f32)
                    + jnp.dot(w1s_ref[...], e, preferred_element_type=f32),
                    0.0)
    h = jnp.maximum(jnp.dot(w2_ref[...], h, preferred_element_type=f32), 0.0)
    h = jnp.maximum(jnp.dot(w3_ref[...], h, preferred_element_type=f32), 0.0)
    deep = jnp.dot(wo_ref[...], h, preferred_element_type=f32)
    out_ref[...] = c_ref[...] + first + second + deep


def _tc_compute(dT, embT, linT, wd, wl, gT, w1d, w1s, w2, w3, wo, cconst):
    nblk = B // BLK

    def full(shape):
        return pl.BlockSpec(shape, lambda i: (0, 0))

    return pl.pallas_call(
        _tc_body,
        grid=(nblk,),
        in_specs=[
            pl.BlockSpec((D, BLK), lambda i: (0, i)),
            pl.BlockSpec((R, BLK), lambda i: (0, i)),
            pl.BlockSpec((F, BLK), lambda i: (0, i)),
            full((K, D)),
            full((1, D)),
            full((K, R)),
            full((512, K)),
            full((512, R)),
            full((256, 512)),
            full((128, 256)),
            full((1, 128)),
            full((1, 1)),
        ],
        out_specs=pl.BlockSpec((1, BLK), lambda i: (0, i)),
        out_shape=jax.ShapeDtypeStruct((1, B), jnp.float32),
    )(dT, embT, linT, wd, wl, gT, w1d, w1s, w2, w3, wo, cconst)


_GT_NP = np.tile(np.eye(K, dtype=np.float32), (1, F))


def kernel(d0, d1, d2, d3, d4, d5, d6, d7, d8, d9, d10, d11, d12,
           s0, s1, s2, s3, s4, s5, s6, s7, s8, s9, s10, s11, s12,
           s13, s14, s15, s16, s17, s18, s19, s20, s21, s22, s23, s24, s25,
           W_dense, W_lin, b_lin, emb_tables, lin_tables, W1, W2, W3, Wout,
           bias):
    dT = jnp.stack([d0, d1, d2, d3, d4, d5, d6, d7, d8, d9, d10, d11, d12],
                   axis=0)
    sT = jnp.stack([s0, s1, s2, s3, s4, s5, s6, s7, s8, s9, s10, s11, s12,
                    s13, s14, s15, s16, s17, s18, s19, s20, s21, s22, s23,
                    s24, s25], axis=0)
    tab = emb_tables.transpose(0, 2, 1).reshape(R, V)
    linT = lin_tables.transpose(0, 2, 1).reshape(F, V)

    embT, linsT = _sc_gather(sT, tab, linT)

    cconst = b_lin.reshape(1, 1) + bias.reshape(1, 1)
    outT = _tc_compute(dT, embT, linsT,
                       W_dense, W_lin, jnp.asarray(_GT_NP),
                       W1[:, :K], W1[:, K:], W2, W3, Wout,
                       cconst)
    return outT.reshape(B, 1)
